# baseline (speedup 1.0000x reference)
import jax
import jax.numpy as jnp
from jax.experimental import pallas as pl
from jax.experimental.pallas import tpu as pltpu


def _gcn_fused_kernel(x_ref, adj_ref, W1_ref, b1_ref, W4_ref, b4_ref,
                      out_ref, s1_ref, s4_ref, *, block_rows):
    p = pl.program_id(0)
    i = pl.program_id(1)

    @pl.when(jnp.logical_and(p == 0, i == 0))
    def _compute_support1():
        s1_ref[...] = jnp.dot(x_ref[...], W1_ref[...],
                              preferred_element_type=jnp.float32)

    @pl.when(p == 0)
    def _phase_aggregate1():
        h = jnp.dot(adj_ref[...], s1_ref[...],
                    preferred_element_type=jnp.float32) + b1_ref[...]
        h = jnp.maximum(h, 0.0)
        s4_ref[pl.ds(i * block_rows, block_rows), :] = jnp.dot(
            h, W4_ref[...], preferred_element_type=jnp.float32)

    @pl.when(p == 1)
    def _phase_aggregate2():
        o = jnp.dot(adj_ref[...], s4_ref[...],
                    preferred_element_type=jnp.float32) + b4_ref[...]
        m = jnp.max(o, axis=1, keepdims=True)
        lse = jnp.log(jnp.sum(jnp.exp(o - m), axis=1, keepdims=True)) + m
        out_ref[...] = o - lse


def kernel(x, adj, W1, b1, W4, b4):
    n, nfeat = x.shape
    nhid = W1.shape[1]
    nclass = W4.shape[1]

    block_rows = 400 if n % 400 == 0 else 8
    nb = n // block_rows

    b1_2d = b1.reshape(1, nhid)
    b4_2d = b4.reshape(1, nclass)

    import functools
    body = functools.partial(_gcn_fused_kernel, block_rows=block_rows)

    out = pl.pallas_call(
        body,
        grid=(2, nb),
        in_specs=[
            pl.BlockSpec((n, nfeat), lambda p, i: (0, 0)),
            pl.BlockSpec((block_rows, n), lambda p, i: (i, 0)),
            pl.BlockSpec((nfeat, nhid), lambda p, i: (0, 0)),
            pl.BlockSpec((1, nhid), lambda p, i: (0, 0)),
            pl.BlockSpec((nhid, nclass), lambda p, i: (0, 0)),
            pl.BlockSpec((1, nclass), lambda p, i: (0, 0)),
        ],
        out_specs=pl.BlockSpec((block_rows, nclass), lambda p, i: (i, 0)),
        out_shape=jax.ShapeDtypeStruct((n, nclass), jnp.float32),
        scratch_shapes=[
            pltpu.VMEM((n, nhid), jnp.float32),
            pltpu.VMEM((n, nclass), jnp.float32),
        ],
        compiler_params=pltpu.CompilerParams(
            dimension_semantics=("arbitrary", "arbitrary"),
        ),
    )(x, adj, W1, b1_2d, W4, b4_2d)
    return out
